# baseline (speedup 1.0000x reference)
import functools
import math

import jax
import jax.numpy as jnp
from jax import lax
from jax.experimental import pallas as pl
from jax.experimental.pallas import tpu as pltpu
from jax.experimental.pallas import tpu_sc as plsc

NUM_EMB = 1000000
DIM = 64
SCALE = math.sqrt(DIM)

_info = plsc.get_sparse_core_info()
NC, NS, L = _info.num_cores, _info.num_subcores, _info.num_lanes
NW = NC * NS

IB = 128
D8 = DIM // 8
BLK = 8 * IB
SG = 4
GR = SG * IB
RU = 16


def _make_kernel(S, NI):
    assert NI == NW and S % SG == 0 and IB % RU == 0
    KMAX = S // SG
    mesh = plsc.VectorSubcoreMesh(core_axis_name="c", subcore_axis_name="s")

    @functools.partial(
        pl.kernel,
        mesh=mesh,
        out_type=jax.ShapeDtypeStruct((S, D8, NI, BLK), jnp.float32),
        scratch_types=[
            pltpu.VMEM((2, GR), jnp.int32),
            pltpu.VMEM((2, GR, DIM), jnp.float32),
            pltpu.VMEM((SG, D8, BLK), jnp.float32),
            pltpu.SemaphoreType.DMA((2,)),
            pltpu.SemaphoreType.DMA((2,)),
            pltpu.SemaphoreType.DMA,
        ],
        compiler_params=pltpu.CompilerParams(
            use_tc_tiling_on_sc=False, needs_layout_passes=False
        ),
    )
    def k(x_hbm, table_hbm, out_hbm, idx_v, rows_v, obuf_v, isem, gsem, ssem):
        w = lax.axis_index("s") * NC + lax.axis_index("c")
        col = w * IB

        def load_idx(kk, b):
            for sl in range(SG):
                pltpu.make_async_copy(
                    x_hbm.at[kk * SG + sl, pl.ds(col, IB)],
                    idx_v.at[b, pl.ds(sl * IB, IB)],
                    isem.at[b],
                ).start()

        def wait_idx(b):
            for _ in range(SG):
                pltpu.make_async_copy(
                    x_hbm.at[0, pl.ds(0, IB)],
                    idx_v.at[0, pl.ds(0, IB)],
                    isem.at[b],
                ).wait()

        def start_gather(b):
            pltpu.make_async_copy(
                table_hbm.at[idx_v.at[b]], rows_v.at[b], gsem.at[b]
            ).start()

        load_idx(0, 0)
        load_idx(1, 1)
        wait_idx(0)
        start_gather(0)

        lanes = lax.iota(jnp.int32, L)

        def item(kk, carry):
            b = kk % 2
            bn = (kk + 1) % 2

            @pl.when(kk + 1 < KMAX)
            def _():
                wait_idx(bn)
                start_gather(bn)

            pltpu.make_async_copy(
                table_hbm.at[idx_v.at[0]], rows_v.at[b], gsem.at[b]
            ).wait()

            @pl.when(kk + 2 < KMAX)
            def _():
                load_idx(kk + 2, b)

            @pl.when(kk > 0)
            def _():
                pltpu.make_async_copy(
                    obuf_v, out_hbm.at[pl.ds(0, SG), :, 0], ssem
                ).wait()

            rows = rows_v.at[b]

            @plsc.parallel_loop(0, GR // L, 1, unroll=2)
            def _(rb):
                rloc = rb * L + lanes
                slv = (rb >> 3) + 0 * lanes
                rmod = (rb & 7) * L + lanes
                for d0 in range(L):
                    dd = (d0 + lanes) & 15
                    maj0 = dd >> 3
                    min0 = ((dd & 7) << 7) + rmod
                    for d16 in range(DIM // L):
                        colvec = dd + d16 * L
                        v = plsc.load_gather(rows, [rloc, colvec])
                        plsc.store_scatter(
                            obuf_v, [slv, maj0 + d16 * 2, min0], v * SCALE
                        )

            pltpu.make_async_copy(
                obuf_v, out_hbm.at[pl.ds(kk * SG, SG), :, w], ssem
            ).start()
            return carry

        lax.fori_loop(0, KMAX, item, 0)
        pltpu.make_async_copy(
            obuf_v, out_hbm.at[pl.ds(0, SG), :, 0], ssem
        ).wait()

    return k


@jax.jit
def kernel(x, table):
    NB, SEQ = x.shape
    xt = jnp.asarray(x, jnp.int32).T
    out4 = _make_kernel(SEQ, NB // IB)(xt, table)
    out5 = out4.reshape(SEQ, D8, NB // IB, 8, IB)
    out = out5.transpose(2, 4, 0, 1, 3).reshape(NB, SEQ, DIM)
    return out


# speedup vs baseline: 1.0482x; 1.0482x over previous
import functools
import math

import jax
import jax.numpy as jnp
from jax import lax
from jax.experimental import pallas as pl
from jax.experimental.pallas import tpu as pltpu
from jax.experimental.pallas import tpu_sc as plsc

NUM_EMB = 1000000
DIM = 64
SCALE = math.sqrt(DIM)

_info = plsc.get_sparse_core_info()
NC, NS, L = _info.num_cores, _info.num_subcores, _info.num_lanes
NW = NC * NS

NBUF = 4
UNROLL = 8


def _make_kernel(B, C):
    b_per_w = B // NW
    n_chunks = b_per_w // C
    assert b_per_w % C == 0 and n_chunks % NBUF == 0 and C % UNROLL == 0
    n_outer = n_chunks // NBUF
    mesh = plsc.VectorSubcoreMesh(core_axis_name="c", subcore_axis_name="s")

    @functools.partial(
        pl.kernel,
        mesh=mesh,
        out_type=jax.ShapeDtypeStruct((B, DIM), jnp.float32),
        scratch_types=[
            pltpu.VMEM((b_per_w,), jnp.int32),
            pltpu.VMEM((NBUF, C, DIM), jnp.float32),
            pltpu.SemaphoreType.DMA((NBUF,)),
            pltpu.SemaphoreType.DMA((NBUF,)),
        ],
        compiler_params=pltpu.CompilerParams(
            use_tc_tiling_on_sc=False, needs_layout_passes=False
        ),
    )
    def k(x_hbm, table_hbm, out_hbm, idx_v, rows_v, gsem, ssem):
        wid = lax.axis_index("s") * NC + lax.axis_index("c")
        base = wid * b_per_w
        pltpu.sync_copy(x_hbm.at[pl.ds(base, b_per_w)], idx_v)

        def start_gather(g, b):
            pltpu.make_async_copy(
                table_hbm.at[idx_v.at[pl.ds(g * C, C)]],
                rows_v.at[b],
                gsem.at[b],
            ).start()

        start_gather(0, 0)
        start_gather(1, 1)

        def outer(o, carry):
            for j in range(NBUF):
                g = o * NBUF + j

                bn = (j + 2) % NBUF

                @pl.when(g >= 2)
                def _():
                    pltpu.make_async_copy(
                        rows_v.at[bn], out_hbm.at[pl.ds(0, C)], ssem.at[bn]
                    ).wait()

                @pl.when(g + 2 < n_chunks)
                def _():
                    start_gather(g + 2, bn)

                pltpu.make_async_copy(
                    table_hbm.at[idx_v.at[pl.ds(0, C)]],
                    rows_v.at[j],
                    gsem.at[j],
                ).wait()

                rows = rows_v.at[j]

                def scale(r0, c2):
                    for u in range(UNROLL):
                        for c4 in range(DIM // L):
                            sl = pl.ds(c4 * L, L)
                            rows[r0 * UNROLL + u, sl] = (
                                rows[r0 * UNROLL + u, sl] * SCALE
                            )
                    return c2

                lax.fori_loop(0, C // UNROLL, scale, 0, unroll=False)
                pltpu.make_async_copy(
                    rows_v.at[j],
                    out_hbm.at[pl.ds(base + g * C, C)],
                    ssem.at[j],
                ).start()
            return carry

        lax.fori_loop(0, n_outer, outer, 0)

        for j in (NBUF - 2, NBUF - 1):
            pltpu.make_async_copy(
                rows_v.at[j], out_hbm.at[pl.ds(0, C)], ssem.at[j]
            ).wait()

    return k


@jax.jit
def kernel(x, table):
    B = x.shape[0] * x.shape[1]
    flat = jnp.asarray(x, jnp.int32).reshape(B)
    out = _make_kernel(B, 400)(flat, table)
    return out.reshape(x.shape[0], x.shape[1], DIM)
